# Initial kernel scaffold; baseline (speedup 1.0000x reference)
import jax
import jax.numpy as jnp
from jax.experimental import pallas as pl


def kernel(features, edge_metapath_indices, edge_index, W_r, b_r, W_fc1, b_fc1, W_fc2):
    raise NotImplementedError("write your pallas kernel here")


# speedup vs baseline: 15.6470x; 15.6470x over previous
import functools

import jax
import jax.numpy as jnp
from jax import lax
from jax.experimental import pallas as pl
from jax.experimental.pallas import tpu as pltpu
from jax.experimental.pallas import tpu_sc as plsc

N = 10000
E = 160000
L = 3
H = 8
OUT = 16
D = 128

NCORE = 2
NSUB = 16
NWORK = NCORE * NSUB
CH = 128
EP = 163840
PER = EP // NWORK
NCHUNK = PER // CH
NPAD = 10240
ROWS = NPAD // NSUB

_F32 = jnp.float32
_HIGH = jax.lax.Precision.HIGHEST

_mesh = plsc.VectorSubcoreMesh(core_axis_name="c", subcore_axis_name="s")


def _dot(a, b):
    return jax.lax.dot_general(a, b, (((1,), (0,)), ((), ())),
                               precision=_HIGH, preferred_element_type=_F32)


def _tc_matmul(x, w):
    def body(x_ref, w_ref, o_ref):
        o_ref[...] = _dot(x_ref[...], w_ref[...])

    return pl.pallas_call(
        body,
        grid=(NPAD // 1024,),
        in_specs=[pl.BlockSpec((1024, D), lambda i: (i, 0)),
                  pl.BlockSpec((D, D), lambda i: (0, 0))],
        out_specs=pl.BlockSpec((1024, D), lambda i: (i, 0)),
        out_shape=jax.ShapeDtypeStruct((NPAD, D), _F32),
    )(x, w)


def _sc_gather(carr, G, feat):
    @functools.partial(
        pl.kernel,
        mesh=_mesh,
        out_type=(jax.ShapeDtypeStruct((EP, D), _F32),
                  jax.ShapeDtypeStruct((EP, D), _F32)),
        scratch_types=[
            pltpu.VMEM((4, CH), jnp.int32),
            pltpu.VMEM((CH, D), _F32),
            pltpu.VMEM((CH, D), _F32),
            pltpu.VMEM((CH, D), _F32),
            pltpu.VMEM((CH, D), _F32),
            pltpu.SemaphoreType.DMA,
        ],
    )
    def k(carr_hbm, G_hbm, feat_hbm, gsum_hbm, nd_hbm, idxv, g0, g1, g2, ndv, sem):
        wid = lax.axis_index("c") * NSUB + lax.axis_index("s")

        def chunk(i, carry):
            base = wid * PER + i * CH
            pltpu.sync_copy(carr_hbm.at[wid * NCHUNK + i], idxv)
            a0 = pltpu.async_copy(G_hbm.at[idxv.at[0]], g0, sem)
            a1 = pltpu.async_copy(G_hbm.at[idxv.at[1]], g1, sem)
            a2 = pltpu.async_copy(G_hbm.at[idxv.at[2]], g2, sem)
            a3 = pltpu.async_copy(feat_hbm.at[idxv.at[3]], ndv, sem)
            a0.wait()
            a1.wait()
            a2.wait()
            a3.wait()

            def row(r, c2):
                for cb in range(D // 16):
                    sl = pl.ds(cb * 16, 16)
                    g0[r, sl] = g0[r, sl] + g1[r, sl] + g2[r, sl]
                return c2

            lax.fori_loop(0, CH, row, 0)
            pltpu.sync_copy(g0, gsum_hbm.at[pl.ds(base, CH)])
            pltpu.sync_copy(ndv, nd_hbm.at[pl.ds(base, CH)])
            return carry

        lax.fori_loop(0, NCHUNK, chunk, 0)

    return k(carr, G, feat)


def _tc_edge(gsum, nd, br):
    BE = 2048

    def body(g_ref, n_ref, b_ref, o_ref):
        h = g_ref[...] * (1.0 / 3.0) + b_ref[...]
        rr = lax.broadcasted_iota(jnp.int32, (D, H), 0) // OUT
        cc = lax.broadcasted_iota(jnp.int32, (D, H), 1)
        S = (rr == cc).astype(_F32)
        rr2 = lax.broadcasted_iota(jnp.int32, (H, D), 0)
        cc2 = lax.broadcasted_iota(jnp.int32, (H, D), 1) // OUT
        ST = (rr2 == cc2).astype(_F32)
        ss = _dot(h * h, S)
        denom = jnp.sqrt(ss) + 1e-12
        sim = _dot(h * n_ref[...], S) / denom
        m = jnp.max(sim, axis=1, keepdims=True)
        e = jnp.exp(sim - m)
        a = e / jnp.sum(e, axis=1, keepdims=True)
        coef = a / denom
        o_ref[...] = h * _dot(coef, ST)

    return pl.pallas_call(
        body,
        grid=(EP // BE,),
        in_specs=[pl.BlockSpec((BE, D), lambda i: (i, 0)),
                  pl.BlockSpec((BE, D), lambda i: (i, 0)),
                  pl.BlockSpec((1, D), lambda i: (0, 0))],
        out_specs=pl.BlockSpec((BE, D), lambda i: (i, 0)),
        out_shape=jax.ShapeDtypeStruct((EP, D), _F32),
    )(gsum, nd, br)


def _sc_scatter(msg, dstp, zeros):
    @functools.partial(
        pl.kernel,
        mesh=_mesh,
        out_type=jax.ShapeDtypeStruct((NCORE, NPAD, D), _F32),
        scratch_types=[
            pltpu.VMEM((CH,), jnp.int32),
            pltpu.VMEM((CH, D), _F32),
            pltpu.VMEM_SHARED((NPAD, D), _F32),
            pltpu.SemaphoreType.DMA,
        ],
    )
    def k(msg_hbm, dst_hbm, z_hbm, part_hbm, dstv, msgv, acc, sem):
        cid = lax.axis_index("c")
        sid = lax.axis_index("s")
        wid = cid * NSUB + sid
        pltpu.sync_copy(z_hbm.at[pl.ds(sid * ROWS, ROWS)],
                        acc.at[pl.ds(sid * ROWS, ROWS)])
        plsc.subcore_barrier()

        def chunk(i, carry):
            base = wid * PER + i * CH
            pltpu.sync_copy(dst_hbm.at[pl.ds(base, CH)], dstv)
            pltpu.sync_copy(msg_hbm.at[pl.ds(base, CH)], msgv)
            pltpu.sync_copy(msgv, acc.at[dstv], add=True)
            return carry

        lax.fori_loop(0, NCHUNK, chunk, 0)
        plsc.subcore_barrier()
        pltpu.sync_copy(acc.at[pl.ds(sid * ROWS, ROWS)],
                        part_hbm.at[cid, pl.ds(sid * ROWS, ROWS)])

    return k(msg, dstp, zeros)


def _tc_add(p):
    def body(p_ref, o_ref):
        o_ref[...] = p_ref[0] + p_ref[1]

    return pl.pallas_call(
        body,
        grid=(NPAD // 1280,),
        in_specs=[pl.BlockSpec((2, 1280, D), lambda i: (0, i, 0))],
        out_specs=pl.BlockSpec((1280, D), lambda i: (i, 0)),
        out_shape=jax.ShapeDtypeStruct((NPAD, D), _F32),
    )(p)


def kernel(features, edge_metapath_indices, edge_index, W_r, b_r, W_fc1, b_fc1, W_fc2):
    featpad = jnp.pad(features, ((0, NPAD - N), (0, 0)))
    G = _tc_matmul(featpad, W_r)

    dst = edge_index[1]
    dstp = jnp.pad(dst, (0, EP - E), constant_values=N)
    idxp = jnp.pad(edge_metapath_indices.T, ((0, 0), (0, EP - E)))
    cidx = jnp.concatenate([idxp, dstp[None]], axis=0)
    carr = cidx.reshape(4, EP // CH, CH).transpose(1, 0, 2)

    gsum, nd = _sc_gather(carr, G, featpad)
    msg = _tc_edge(gsum, nd, jnp.reshape(b_r, (1, D)))
    parts = _sc_scatter(msg, dstp, jnp.zeros((NPAD, D), _F32))
    out = _tc_add(parts)
    return out[:N]
